# baseline (speedup 1.0000x reference)
import functools

import jax
import jax.numpy as jnp
from jax import lax
from jax.experimental import pallas as pl
from jax.experimental.pallas import tpu as pltpu
from jax.experimental.pallas import tpu_sc as plsc

NUM_CODES = 8192
CODE_DIM = 256
N_ROWS = 16 * 1024
BLOCK_ROWS = 256
GRID = N_ROWS // BLOCK_ROWS

_SC_INFO = plsc.get_sparse_core_info()
_NC, _NS = _SC_INFO.num_cores, _SC_INFO.num_subcores
_NW = _NC * _NS
_ROWS_PER_W = N_ROWS // _NW
_CHUNK = 128
_NCHUNK = _ROWS_PER_W // _CHUNK


_CHUNK_BOUNDS = ((0, 2736), (2736, 5472), (5472, 8192))


def _dist_argmin_kernel(z_ref, cb_ref, idx_ref, minv_ref, dist_ref):
    zb = z_ref[...]
    cbv = cb_ref[...]
    mm = lax.dot_general(zb, cbv, (((1,), (1,)), ((), ())),
                         preferred_element_type=jnp.float32)
    s_z = jnp.sum(zb * zb, axis=1, keepdims=True)
    s_c = jnp.sum(cbv * cbv, axis=1)
    dist_ref[...] = (s_z + s_c[None, :]) - 2.0 * mm
    dist = dist_ref[...]
    inf = jnp.float32(jnp.inf)

    l1 = lax.broadcasted_iota(jnp.int32, (BLOCK_ROWS, 128), 1)

    def slice_min(g, keep=None):
        base = g * 128
        block = dist[:, base:base + 128]
        bm = block if keep is None else jnp.where(keep, block, inf)
        pm = jnp.min(bm, axis=1)
        pi = jnp.min(jnp.where(bm == pm[:, None], l1 + base, NUM_CODES),
                     axis=1).astype(jnp.int32)
        return pm, pi

    def build_chunk(pieces):
        pmin, pidx = pieces[0]
        for pm, pi in pieces[1:]:
            take = pm < pmin
            pidx = jnp.where(take, pi, pidx)
            pmin = jnp.where(take, pm, pmin)
        return pmin, pidx

    c0 = [slice_min(g) for g in range(0, 21)] + [slice_min(21, l1 < 48)]
    c1 = ([slice_min(21, l1 >= 48)]
          + [slice_min(g) for g in range(22, 42)]
          + [slice_min(42, l1 < 96)])
    c2 = [slice_min(42, l1 >= 96)] + [slice_min(g) for g in range(43, 64)]
    chunks = [build_chunk(c0), build_chunk(c1), build_chunk(c2)]

    acc_v = jnp.full((BLOCK_ROWS,), inf, jnp.float32)
    acc_i = jnp.zeros((BLOCK_ROWS,), jnp.int32)
    val_i = jnp.full((BLOCK_ROWS,), inf, jnp.float32)
    for pmin, pidx in chunks:
        lt = acc_v < pmin
        nan = acc_v != acc_v
        keep_v = jnp.logical_or(lt, nan)
        keep_i = jnp.logical_or(keep_v, jnp.logical_and(acc_v == pmin,
                                                        acc_i < pidx))
        acc_i = jnp.where(keep_i, acc_i, pidx)
        val_i = jnp.where(keep_i, val_i, pmin)
        acc_v = jnp.where(keep_v, acc_v, pmin
                          ).astype(jnp.bfloat16).astype(jnp.float32)
    idx_ref[...] = acc_i
    minv_ref[...] = val_i


def _loss_kernel(minv_ref, loss_ref):
    total = jnp.sum(minv_ref[...])
    m = total * (1.0 / (N_ROWS * CODE_DIM))
    loss_ref[...] = jnp.full((1, 1), m + 0.25 * m, jnp.float32)


def _sc_gather_kernel(cb_hbm, idx_hbm, out_hbm, idx_v, rows_v, sem):
    wid = lax.axis_index("s") * _NC + lax.axis_index("c")
    base = wid * _ROWS_PER_W
    for k in range(_NCHUNK):
        start = base + k * _CHUNK
        pltpu.sync_copy(idx_hbm.at[pl.ds(start, _CHUNK)], idx_v)
        pltpu.async_copy(cb_hbm.at[idx_v], rows_v, sem).wait()
        pltpu.sync_copy(rows_v, out_hbm.at[pl.ds(start, _CHUNK)])


@functools.partial(jax.jit, static_argnames=("interpret",))
def kernel(z, codebook, interpret=False):
    z_flat = z.reshape(-1, CODE_DIM)

    idx, minv = pl.pallas_call(
        _dist_argmin_kernel,
        grid=(GRID,),
        in_specs=[
            pl.BlockSpec((BLOCK_ROWS, CODE_DIM), lambda i: (i, 0)),
            pl.BlockSpec((NUM_CODES, CODE_DIM), lambda i: (0, 0)),
        ],
        out_specs=[
            pl.BlockSpec((BLOCK_ROWS,), lambda i: (i,)),
            pl.BlockSpec((BLOCK_ROWS,), lambda i: (i,)),
        ],
        out_shape=[
            jax.ShapeDtypeStruct((N_ROWS,), jnp.int32),
            jax.ShapeDtypeStruct((N_ROWS,), jnp.float32),
        ],
        scratch_shapes=[pltpu.VMEM((BLOCK_ROWS, NUM_CODES), jnp.float32)],
        interpret=interpret,
    )(z_flat, codebook)

    loss = pl.pallas_call(
        _loss_kernel,
        out_shape=jax.ShapeDtypeStruct((1, 1), jnp.float32),
        interpret=interpret,
    )(minv.reshape(128, 128))

    sc_gather = pl.kernel(
        _sc_gather_kernel,
        out_type=jax.ShapeDtypeStruct((N_ROWS, CODE_DIM), jnp.float32),
        mesh=plsc.VectorSubcoreMesh(core_axis_name="c", subcore_axis_name="s"),
        scratch_types=[
            pltpu.VMEM((_CHUNK,), jnp.int32),
            pltpu.VMEM((_CHUNK, CODE_DIM), jnp.float32),
            pltpu.SemaphoreType.DMA,
        ],
    )
    out = sc_gather(codebook, idx)

    return (out.reshape(z.shape), loss[0, 0], idx.reshape(z.shape[:-1]))


# speedup vs baseline: 3.3396x; 3.3396x over previous
import functools

import jax
import jax.numpy as jnp
from jax import lax
from jax.experimental import pallas as pl
from jax.experimental.pallas import tpu as pltpu
from jax.experimental.pallas import tpu_sc as plsc

NUM_CODES = 8192
CODE_DIM = 256
N_ROWS = 16 * 1024
BLOCK_ROWS = 256
GRID = N_ROWS // BLOCK_ROWS

_SC_INFO = plsc.get_sparse_core_info()
_NC, _NS = _SC_INFO.num_cores, _SC_INFO.num_subcores
_NW = _NC * _NS
_ROWS_PER_W = N_ROWS // _NW
_CHUNK = 128
_NCHUNK = _ROWS_PER_W // _CHUNK


_CHUNK_BOUNDS = ((0, 2736), (2736, 5472), (5472, 8192))


def _dist_argmin_kernel(z_ref, cb_ref, idx_ref, minv_ref, dist_ref):
    zb = z_ref[...]
    cbv = cb_ref[...]
    mm = lax.dot_general(zb, cbv, (((1,), (1,)), ((), ())),
                         preferred_element_type=jnp.float32)
    s_z = jnp.sum(zb * zb, axis=1, keepdims=True)
    s_c = jnp.sum(cbv * cbv, axis=1)
    dist_ref[...] = (s_z + s_c[None, :]) - 2.0 * mm
    dist = dist_ref[...]

    acc_v = jnp.full((BLOCK_ROWS,), jnp.inf, jnp.float32)
    acc_i = jnp.zeros((BLOCK_ROWS,), jnp.int32)
    val_i = jnp.full((BLOCK_ROWS,), jnp.inf, jnp.float32)
    for (lo, hi), (slo, shi) in zip(_CHUNK_BOUNDS,
                                    ((0, 2816), (2688, 5504), (5376, 8192))):
        sub = dist[:, slo:shi]
        li = lax.broadcasted_iota(jnp.int32, (BLOCK_ROWS, shi - slo), 1) + slo
        in_chunk = jnp.logical_and(li >= lo, li < hi)
        blk = jnp.where(in_chunk, sub, jnp.inf)
        pmin = jnp.min(blk, axis=1)
        pidx = jnp.min(jnp.where(blk == pmin[:, None], li, NUM_CODES),
                       axis=1).astype(jnp.int32)
        lt = acc_v < pmin
        nan = acc_v != acc_v
        keep_v = jnp.logical_or(lt, nan)
        keep_i = jnp.logical_or(keep_v, jnp.logical_and(acc_v == pmin,
                                                        acc_i < pidx))
        acc_i = jnp.where(keep_i, acc_i, pidx)
        val_i = jnp.where(keep_i, val_i, pmin)
        acc_v = jnp.where(keep_v, acc_v, pmin
                          ).astype(jnp.bfloat16).astype(jnp.float32)
    idx_ref[...] = acc_i
    minv_ref[...] = val_i


def _loss_kernel(minv_ref, loss_ref):
    total = jnp.sum(minv_ref[...])
    m = total * (1.0 / (N_ROWS * CODE_DIM))
    loss_ref[...] = jnp.full((1, 1), m + 0.25 * m, jnp.float32)


def _sc_gather_kernel(cb_hbm, idx_hbm, out_hbm, idx_v, rows_v, sem):
    wid = lax.axis_index("s") * _NC + lax.axis_index("c")
    base = wid * _ROWS_PER_W
    for k in range(_NCHUNK):
        start = base + k * _CHUNK
        pltpu.sync_copy(idx_hbm.at[pl.ds(start, _CHUNK)], idx_v)
        pltpu.async_copy(cb_hbm.at[idx_v], rows_v, sem).wait()
        pltpu.sync_copy(rows_v, out_hbm.at[pl.ds(start, _CHUNK)])


@functools.partial(jax.jit, static_argnames=("interpret",))
def kernel(z, codebook, interpret=False):
    z_flat = z.reshape(-1, CODE_DIM)

    idx, minv = pl.pallas_call(
        _dist_argmin_kernel,
        grid=(GRID,),
        in_specs=[
            pl.BlockSpec((BLOCK_ROWS, CODE_DIM), lambda i: (i, 0)),
            pl.BlockSpec((NUM_CODES, CODE_DIM), lambda i: (0, 0)),
        ],
        out_specs=[
            pl.BlockSpec((BLOCK_ROWS,), lambda i: (i,)),
            pl.BlockSpec((BLOCK_ROWS,), lambda i: (i,)),
        ],
        out_shape=[
            jax.ShapeDtypeStruct((N_ROWS,), jnp.int32),
            jax.ShapeDtypeStruct((N_ROWS,), jnp.float32),
        ],
        scratch_shapes=[pltpu.VMEM((BLOCK_ROWS, NUM_CODES), jnp.float32)],
        interpret=interpret,
    )(z_flat, codebook)

    loss = pl.pallas_call(
        _loss_kernel,
        out_shape=jax.ShapeDtypeStruct((1, 1), jnp.float32),
        interpret=interpret,
    )(minv.reshape(128, 128))

    sc_gather = pl.kernel(
        _sc_gather_kernel,
        out_type=jax.ShapeDtypeStruct((N_ROWS, CODE_DIM), jnp.float32),
        mesh=plsc.VectorSubcoreMesh(core_axis_name="c", subcore_axis_name="s"),
        scratch_types=[
            pltpu.VMEM((_CHUNK,), jnp.int32),
            pltpu.VMEM((_CHUNK, CODE_DIM), jnp.float32),
            pltpu.SemaphoreType.DMA,
        ],
    )
    out = sc_gather(codebook, idx)

    return (out.reshape(z.shape), loss[0, 0], idx.reshape(z.shape[:-1]))
